# Initial kernel scaffold; baseline (speedup 1.0000x reference)
import functools

import jax
import jax.numpy as jnp
from jax import lax
from jax.experimental import pallas as pl
from jax.experimental.pallas import tpu as pltpu
from jax.experimental.pallas import tpu_sc as plsc

NUM_ENTS = 10000
NUM_RELS = 200
H = 128
T = 3
E = 320000
SLOPE = (1.0 / 8.0 + 1.0 / 3.0) / 2.0

NC = 2
NS = 16
NW = NC * NS
CHUNK = 128
CH = 80
EPW = CH * CHUNK
EPAD = EPW * NW
AGG_R = NUM_ENTS + 16
REL_R = 2 * NUM_RELS + 16


def _l2norm(x):
    n = jnp.sqrt(jnp.sum(x * x, axis=-1, keepdims=True))
    return x / jnp.maximum(n, 1e-12)


def _rrelu(x):
    return jnp.where(x >= 0, x, x * SLOPE)


def _dot(a, b):
    return lax.dot_general(a, b, (((1,), (0,)), ((), ())),
                           preferred_element_type=jnp.float32)


def _dot_t(a, b):
    return lax.dot_general(a, b, (((1,), (1,)), ((), ())),
                           preferred_element_type=jnp.float32)


BLK = 2000


def _tc_prep(emb, w0):
    def body(emb_r, w0_r, h_r, hw_r):
        h = _l2norm(emb_r[...])
        h_r[...] = h
        hw_r[...] = _dot(h, w0_r[...])

    return pl.pallas_call(
        body,
        grid=(NUM_ENTS // BLK,),
        in_specs=[pl.BlockSpec((BLK, H), lambda i: (i, 0)),
                  pl.BlockSpec((H, H), lambda i: (0, 0))],
        out_specs=[pl.BlockSpec((BLK, H), lambda i: (i, 0)),
                   pl.BlockSpec((BLK, H), lambda i: (i, 0))],
        out_shape=[jax.ShapeDtypeStruct((NUM_ENTS, H), jnp.float32),
                   jax.ShapeDtypeStruct((NUM_ENTS, H), jnp.float32)],
    )(emb, w0)


def _tc_relprep(emb_rel, w0, w1):
    def body(er_r, w0_r, w1_r, o0_r, o1_r):
        er = er_r[...]
        o0_r[...] = _dot(er, w0_r[...])
        o1_r[...] = _dot(er, w1_r[...])

    R = 2 * NUM_RELS
    return pl.pallas_call(
        body,
        grid=(1,),
        in_specs=[pl.BlockSpec((R, H), lambda i: (0, 0)),
                  pl.BlockSpec((H, H), lambda i: (0, 0)),
                  pl.BlockSpec((H, H), lambda i: (0, 0))],
        out_specs=[pl.BlockSpec((R, H), lambda i: (0, 0)),
                   pl.BlockSpec((R, H), lambda i: (0, 0))],
        out_shape=[jax.ShapeDtypeStruct((R, H), jnp.float32),
                   jax.ShapeDtypeStruct((R, H), jnp.float32)],
    )(emb_rel, w0, w1)


def _tc_mid(agg_a, agg_b, deg_a, deg_b, h, lw, ew, w_next):
    def body(aa_r, ab_r, da_r, db_r, h_r, lw_r, ew_r, wn_r, cur_r, hw_r):
        deg = da_r[...][:, :1] + db_r[...][:, :1]
        norm = 1.0 / jnp.maximum(deg, 1.0)
        h = h_r[...]
        lm = jnp.where(deg > 0, _dot(h, lw_r[...]), _dot(h, ew_r[...]))
        cur = _rrelu((aa_r[...] + ab_r[...]) * norm + lm)
        cur_r[...] = cur
        hw_r[...] = _dot(cur, wn_r[...])

    return pl.pallas_call(
        body,
        grid=(NUM_ENTS // BLK,),
        in_specs=[pl.BlockSpec((BLK, H), lambda i: (i, 0)),
                  pl.BlockSpec((BLK, H), lambda i: (i, 0)),
                  pl.BlockSpec((BLK, 16), lambda i: (i, 0)),
                  pl.BlockSpec((BLK, 16), lambda i: (i, 0)),
                  pl.BlockSpec((BLK, H), lambda i: (i, 0)),
                  pl.BlockSpec((H, H), lambda i: (0, 0)),
                  pl.BlockSpec((H, H), lambda i: (0, 0)),
                  pl.BlockSpec((H, H), lambda i: (0, 0))],
        out_specs=[pl.BlockSpec((BLK, H), lambda i: (i, 0)),
                   pl.BlockSpec((BLK, H), lambda i: (i, 0))],
        out_shape=[jax.ShapeDtypeStruct((NUM_ENTS, H), jnp.float32),
                   jax.ShapeDtypeStruct((NUM_ENTS, H), jnp.float32)],
    )(agg_a, agg_b, deg_a, deg_b, h, lw, ew, w_next)


def _tc_post(agg_a, agg_b, deg_a, deg_b, cur1, h, lw, ew, tgw, tgb, w0):
    def body(aa_r, ab_r, da_r, db_r, c1_r, h_r, lw_r, ew_r, tgw_r, tgb_r,
             w0_r, hn_r, hw_r):
        deg = da_r[...][:, :1] + db_r[...][:, :1]
        norm = 1.0 / jnp.maximum(deg, 1.0)
        c1 = c1_r[...]
        lm = jnp.where(deg > 0, _dot(c1, lw_r[...]), _dot(c1, ew_r[...]))
        cur2 = _rrelu((aa_r[...] + ab_r[...]) * norm + lm)
        ch = _l2norm(cur2)
        h = h_r[...]
        tw = jax.nn.sigmoid(_dot(h, tgw_r[...]) + tgb_r[...][None, :])
        hn = tw * ch + (1.0 - tw) * h
        hn_r[...] = hn
        hw_r[...] = _dot(hn, w0_r[...])

    return pl.pallas_call(
        body,
        grid=(NUM_ENTS // BLK,),
        in_specs=[pl.BlockSpec((BLK, H), lambda i: (i, 0)),
                  pl.BlockSpec((BLK, H), lambda i: (i, 0)),
                  pl.BlockSpec((BLK, 16), lambda i: (i, 0)),
                  pl.BlockSpec((BLK, 16), lambda i: (i, 0)),
                  pl.BlockSpec((BLK, H), lambda i: (i, 0)),
                  pl.BlockSpec((BLK, H), lambda i: (i, 0)),
                  pl.BlockSpec((H, H), lambda i: (0, 0)),
                  pl.BlockSpec((H, H), lambda i: (0, 0)),
                  pl.BlockSpec((H, H), lambda i: (0, 0)),
                  pl.BlockSpec((H,), lambda i: (0,)),
                  pl.BlockSpec((H, H), lambda i: (0, 0))],
        out_specs=[pl.BlockSpec((BLK, H), lambda i: (i, 0)),
                   pl.BlockSpec((BLK, H), lambda i: (i, 0))],
        out_shape=[jax.ShapeDtypeStruct((NUM_ENTS, H), jnp.float32),
                   jax.ShapeDtypeStruct((NUM_ENTS, H), jnp.float32)],
    )(agg_a, agg_b, deg_a, deg_b, cur1, h, lw, ew, tgw, tgb, w0)


def _tc_gru(sr_a, sr_b, ct_a, ct_b, emb_rel, h0, w_ih, b_ih, w_hh, b_hh):
    def body(sa_r, sb_r, ca_r, cb_r, er_r, h0_r, wih_r, bih_r, whh_r, bhh_r,
             out_r):
        cnt = 2.0 * (ca_r[...][:, :1] + cb_r[...][:, :1])
        sr = sa_r[...] + sb_r[...]
        x_in = jnp.where(cnt > 0, sr / jnp.maximum(cnt, 1.0), 0.0)
        er = er_r[...]
        wih = wih_r[...]
        gi = (_dot_t(er, wih[:, :H]) + _dot_t(x_in, wih[:, H:])
              + bih_r[...][None, :])
        h0 = h0_r[...]
        gh = _dot_t(h0, whh_r[...]) + bhh_r[...][None, :]
        r = jax.nn.sigmoid(gi[:, :H] + gh[:, :H])
        z = jax.nn.sigmoid(gi[:, H:2 * H] + gh[:, H:2 * H])
        n = jnp.tanh(gi[:, 2 * H:] + r * gh[:, 2 * H:])
        out_r[...] = _l2norm((1.0 - z) * n + z * h0)

    R = 2 * NUM_RELS
    return pl.pallas_call(
        body,
        grid=(1,),
        in_specs=[pl.BlockSpec((R, H), lambda i: (0, 0)),
                  pl.BlockSpec((R, H), lambda i: (0, 0)),
                  pl.BlockSpec((R, 16), lambda i: (0, 0)),
                  pl.BlockSpec((R, 16), lambda i: (0, 0)),
                  pl.BlockSpec((R, H), lambda i: (0, 0)),
                  pl.BlockSpec((R, H), lambda i: (0, 0)),
                  pl.BlockSpec((3 * H, 2 * H), lambda i: (0, 0)),
                  pl.BlockSpec((3 * H,), lambda i: (0,)),
                  pl.BlockSpec((3 * H, H), lambda i: (0, 0)),
                  pl.BlockSpec((3 * H,), lambda i: (0,))],
        out_specs=pl.BlockSpec((R, H), lambda i: (0, 0)),
        out_shape=jax.ShapeDtypeStruct((R, H), jnp.float32),
    )(sr_a, sr_b, ct_a, ct_b, emb_rel, h0, w_ih, b_ih, w_hh, b_hh)


def _sc_mesh():
    return plsc.VectorSubcoreMesh(core_axis_name="c", subcore_axis_name="s")


def _zero_tables(s, zbuf, agg_sh, deg_sh, sumr_sh=None, cnt_sh=None):
    za = AGG_R // NS
    base = s * za
    off = 0
    for n in (128, 128, 128, 128, za - 512):
        pltpu.sync_copy(zbuf.at[pl.ds(0, n)], agg_sh.at[pl.ds(base + off, n)])
        if deg_sh is not None:
            pltpu.sync_copy(zbuf.at[pl.ds(0, n), pl.ds(0, 16)],
                            deg_sh.at[pl.ds(base + off, n)])
        off += n
    if sumr_sh is not None:
        zr = REL_R // NS
        rb = s * zr
        pltpu.sync_copy(zbuf.at[pl.ds(0, zr)], sumr_sh.at[pl.ds(rb, zr)])
        pltpu.sync_copy(zbuf.at[pl.ds(0, zr), pl.ds(0, 16)],
                        cnt_sh.at[pl.ds(rb, zr)])


def _stream_job(tab_h, gidx, tgt_sh, sidx, rows0, rows1, sem0, sem1):
    @pl.loop(0, CH, step=2)
    def _(j):
        g0 = pltpu.async_copy(tab_h.at[gidx.at[j]], rows0, sem0)
        g1 = pltpu.async_copy(tab_h.at[gidx.at[j + 1]], rows1, sem1)
        g0.wait()
        pltpu.sync_copy(rows0, tgt_sh.at[sidx.at[j]], add=True)
        g1.wait()
        pltpu.sync_copy(rows1, tgt_sh.at[sidx.at[j + 1]], add=True)


def _ones_job(ones, tgt_sh, sidx):
    @pl.loop(0, CH)
    def _(j):
        pltpu.sync_copy(ones, tgt_sh.at[sidx.at[j]], add=True)


def _sc_stage_a(hw1, erw1, hpad, src_i, dst_i, et_i, zeros_h, ones_h):
    out_type = [
        jax.ShapeDtypeStruct((NC, NUM_ENTS, H), jnp.float32),
        jax.ShapeDtypeStruct((NC, NUM_ENTS, 16), jnp.float32),
        jax.ShapeDtypeStruct((NC, 2 * NUM_RELS, H), jnp.float32),
        jax.ShapeDtypeStruct((NC, 2 * NUM_RELS, 16), jnp.float32),
    ]
    scratch = [
        pltpu.VMEM((CH, CHUNK), jnp.int32),
        pltpu.VMEM((CH, CHUNK), jnp.int32),
        pltpu.VMEM((CH, CHUNK), jnp.int32),
        pltpu.VMEM((CHUNK, H), jnp.float32),
        pltpu.VMEM((CHUNK, H), jnp.float32),
        pltpu.VMEM((CHUNK, H), jnp.float32),
        pltpu.VMEM((CHUNK, 16), jnp.float32),
        pltpu.VMEM_SHARED((AGG_R, H), jnp.float32),
        pltpu.VMEM_SHARED((AGG_R, 16), jnp.float32),
        pltpu.VMEM_SHARED((REL_R, H), jnp.float32),
        pltpu.VMEM_SHARED((REL_R, 16), jnp.float32),
        pltpu.SemaphoreType.DMA,
        pltpu.SemaphoreType.DMA,
    ]

    @functools.partial(pl.kernel, out_type=out_type, mesh=_sc_mesh(),
                       scratch_types=scratch)
    def k(hw1_h, erw1_h, hpad_h, src_h, dst_h, et_h, z_h, o_h,
          agg_o, deg_o, sumr_o, cnt_o,
          src_s, dst_s, et_s, rows0, rows1, zbuf, ones,
          agg_sh, deg_sh, sumr_sh, cnt_sh, sem0, sem1):
        c = lax.axis_index("c")
        s = lax.axis_index("s")
        w = s * NC + c
        pltpu.sync_copy(z_h, zbuf)
        pltpu.sync_copy(o_h, ones)
        _zero_tables(s, zbuf, agg_sh, deg_sh, sumr_sh, cnt_sh)
        pltpu.sync_copy(src_h.at[w], src_s)
        pltpu.sync_copy(dst_h.at[w], dst_s)
        pltpu.sync_copy(et_h.at[w], et_s)
        plsc.subcore_barrier()
        _stream_job(hw1_h, src_s, agg_sh, dst_s, rows0, rows1, sem0, sem1)
        _stream_job(erw1_h, et_s, agg_sh, dst_s, rows0, rows1, sem0, sem1)
        _stream_job(hpad_h, src_s, sumr_sh, et_s, rows0, rows1, sem0, sem1)
        _stream_job(hpad_h, dst_s, sumr_sh, et_s, rows0, rows1, sem0, sem1)
        _ones_job(ones, deg_sh, dst_s)
        _ones_job(ones, cnt_sh, et_s)
        plsc.subcore_barrier()
        oa = NUM_ENTS // NS
        pltpu.sync_copy(agg_sh.at[pl.ds(s * oa, oa)],
                        agg_o.at[c].at[pl.ds(s * oa, oa)])
        pltpu.sync_copy(deg_sh.at[pl.ds(s * oa, oa)],
                        deg_o.at[c].at[pl.ds(s * oa, oa)])
        orr = 2 * NUM_RELS // NS
        pltpu.sync_copy(sumr_sh.at[pl.ds(s * orr, orr)],
                        sumr_o.at[c].at[pl.ds(s * orr, orr)])
        pltpu.sync_copy(cnt_sh.at[pl.ds(s * orr, orr)],
                        cnt_o.at[c].at[pl.ds(s * orr, orr)])

    return k(hw1, erw1, hpad, src_i, dst_i, et_i, zeros_h, ones_h)


def _sc_stage_b(hw2, erw2, src_i, dst_i, et_i, zeros_h):
    out_type = jax.ShapeDtypeStruct((NC, NUM_ENTS, H), jnp.float32)
    scratch = [
        pltpu.VMEM((CH, CHUNK), jnp.int32),
        pltpu.VMEM((CH, CHUNK), jnp.int32),
        pltpu.VMEM((CH, CHUNK), jnp.int32),
        pltpu.VMEM((CHUNK, H), jnp.float32),
        pltpu.VMEM((CHUNK, H), jnp.float32),
        pltpu.VMEM((CHUNK, H), jnp.float32),
        pltpu.VMEM_SHARED((AGG_R, H), jnp.float32),
        pltpu.SemaphoreType.DMA,
        pltpu.SemaphoreType.DMA,
    ]

    @functools.partial(pl.kernel, out_type=out_type, mesh=_sc_mesh(),
                       scratch_types=scratch)
    def k(hw2_h, erw2_h, src_h, dst_h, et_h, z_h,
          agg_o,
          src_s, dst_s, et_s, rows0, rows1, zbuf,
          agg_sh, sem0, sem1):
        c = lax.axis_index("c")
        s = lax.axis_index("s")
        w = s * NC + c
        pltpu.sync_copy(z_h, zbuf)
        _zero_tables(s, zbuf, agg_sh, None)
        pltpu.sync_copy(src_h.at[w], src_s)
        pltpu.sync_copy(dst_h.at[w], dst_s)
        pltpu.sync_copy(et_h.at[w], et_s)
        plsc.subcore_barrier()
        _stream_job(hw2_h, src_s, agg_sh, dst_s, rows0, rows1, sem0, sem1)
        _stream_job(erw2_h, et_s, agg_sh, dst_s, rows0, rows1, sem0, sem1)
        plsc.subcore_barrier()
        oa = NUM_ENTS // NS
        pltpu.sync_copy(agg_sh.at[pl.ds(s * oa, oa)],
                        agg_o.at[c].at[pl.ds(s * oa, oa)])

    return k(hw2, erw2, src_i, dst_i, et_i, zeros_h)


def kernel(edge_src, edge_dst, edge_type, dynamic_emb, emb_rel, w_ih, b_ih,
           w_hh, b_hh, time_gate_w, time_gate_b, w_neigh_0, loop_w_0,
           evolve_w_0, w_neigh_1, loop_w_1, evolve_w_1):
    pad = EPAD - E
    srcp = jnp.pad(edge_src, ((0, 0), (0, pad))).reshape(T, NW, CH, CHUNK)
    dstp = jnp.pad(edge_dst, ((0, 0), (0, pad)),
                   constant_values=NUM_ENTS).reshape(T, NW, CH, CHUNK)
    etp = jnp.pad(edge_type, ((0, 0), (0, pad)),
                  constant_values=2 * NUM_RELS).reshape(T, NW, CH, CHUNK)
    zeros_h = jnp.zeros((CHUNK, H), jnp.float32)
    ones_h = jnp.ones((CHUNK, 16), jnp.float32)

    erw1, erw2 = _tc_relprep(emb_rel, w_neigh_0, w_neigh_1)
    erw1p = jnp.pad(erw1, ((0, REL_R - 2 * NUM_RELS), (0, 0)))
    erw2p = jnp.pad(erw2, ((0, REL_R - 2 * NUM_RELS), (0, 0)))

    h, hw1 = _tc_prep(dynamic_emb, w_neigh_0)
    h0 = emb_rel
    hist = []
    for t in range(T):
        hpad = jnp.pad(h, ((0, AGG_R - NUM_ENTS), (0, 0)))
        aggp, degp, sumrp, cntp = _sc_stage_a(
            hw1, erw1p, hpad, srcp[t], dstp[t], etp[t], zeros_h, ones_h)
        cur1, hw2 = _tc_mid(aggp[0], aggp[1], degp[0], degp[1], h,
                            loop_w_0, evolve_w_0, w_neigh_1)
        h0 = _tc_gru(sumrp[0], sumrp[1], cntp[0], cntp[1], emb_rel, h0,
                     w_ih, b_ih, w_hh, b_hh)
        agg2p = _sc_stage_b(hw2, erw2p, srcp[t], dstp[t], etp[t], zeros_h)
        h, hw1 = _tc_post(agg2p[0], agg2p[1], degp[0], degp[1], cur1, h,
                          loop_w_1, evolve_w_1, time_gate_w, time_gate_b,
                          w_neigh_0)
        hist.append(h)
    return jnp.stack(hist, axis=0), h0


# speedup vs baseline: 1.0538x; 1.0538x over previous
import functools

import jax
import jax.numpy as jnp
from jax import lax
from jax.experimental import pallas as pl
from jax.experimental.pallas import tpu as pltpu
from jax.experimental.pallas import tpu_sc as plsc

NUM_ENTS = 10000
NUM_RELS = 200
H = 128
T = 3
E = 320000
SLOPE = (1.0 / 8.0 + 1.0 / 3.0) / 2.0

NC = 2
NS = 16
CHUNK = 128
CH = 160
HCH = CH // 2
EPW = CH * CHUNK
EPAD = EPW * NS
ROWS = 10240
HALF = ROWS // 2
RPC = 5248
JUNK = HALF
RELS = 512
BLK = 2560


def _l2norm(x):
    n = jnp.sqrt(jnp.sum(x * x, axis=-1, keepdims=True))
    return x / jnp.maximum(n, 1e-12)


def _rrelu(x):
    return jnp.where(x >= 0, x, x * SLOPE)


def _dot(a, b):
    return lax.dot_general(a, b, (((1,), (0,)), ((), ())),
                           preferred_element_type=jnp.float32)


def _dot_t(a, b):
    return lax.dot_general(a, b, (((1,), (1,)), ((), ())),
                           preferred_element_type=jnp.float32)


def _map_sp(i):
    return (i // 2, i % 2, 0)


def _map_row(i):
    return (i, 0)


def _map0(i):
    return (0, 0)


def _tc_prep(emb, w0, w1, emb_rel):
    def body(emb_r, w0_r, w1_r, er_r, h_r, hw_r, e1_r, e2_r):
        h = _l2norm(emb_r[...])
        h_r[...] = h
        hw_r[...] = _dot(h, w0_r[...])
        er = er_r[...]
        e1_r[...] = _dot(er, w0_r[...])
        e2_r[...] = _dot(er, w1_r[...])

    return pl.pallas_call(
        body,
        grid=(ROWS // BLK,),
        in_specs=[pl.BlockSpec((BLK, H), _map_row),
                  pl.BlockSpec((H, H), _map0),
                  pl.BlockSpec((H, H), _map0),
                  pl.BlockSpec((RELS, H), _map0)],
        out_specs=[pl.BlockSpec((BLK, H), _map_row),
                   pl.BlockSpec((BLK, H), _map_row),
                   pl.BlockSpec((RELS, H), _map0),
                   pl.BlockSpec((RELS, H), _map0)],
        out_shape=[jax.ShapeDtypeStruct((ROWS, H), jnp.float32),
                   jax.ShapeDtypeStruct((ROWS, H), jnp.float32),
                   jax.ShapeDtypeStruct((RELS, H), jnp.float32),
                   jax.ShapeDtypeStruct((RELS, H), jnp.float32)],
    )(emb, w0, w1, emb_rel)


def _tc_mid(agg, deg, h, lw, ew, w_next):
    def body(ag_r, dg_r, h_r, lw_r, ew_r, wn_r, cur_r, hw_r):
        degv = dg_r[...][0, :, :1]
        norm = 1.0 / jnp.maximum(degv, 1.0)
        hv = h_r[...]
        lm = jnp.where(degv > 0, _dot(hv, lw_r[...]), _dot(hv, ew_r[...]))
        cur = _rrelu(ag_r[...][0] * norm + lm)
        cur_r[...] = cur
        hw_r[...] = _dot(cur, wn_r[...])

    return pl.pallas_call(
        body,
        grid=(ROWS // BLK,),
        in_specs=[pl.BlockSpec((1, BLK, H), _map_sp),
                  pl.BlockSpec((1, BLK, H), _map_sp),
                  pl.BlockSpec((BLK, H), _map_row),
                  pl.BlockSpec((H, H), _map0),
                  pl.BlockSpec((H, H), _map0),
                  pl.BlockSpec((H, H), _map0)],
        out_specs=[pl.BlockSpec((BLK, H), _map_row),
                   pl.BlockSpec((BLK, H), _map_row)],
        out_shape=[jax.ShapeDtypeStruct((ROWS, H), jnp.float32),
                   jax.ShapeDtypeStruct((ROWS, H), jnp.float32)],
    )(agg, deg, h, lw, ew, w_next)


def _tc_post(agg, deg, cur1, h, lw, ew, tgw, tgb, w0):
    def body(ag_r, dg_r, c1_r, h_r, lw_r, ew_r, tgw_r, tgb_r, w0_r,
             hn_r, hw_r):
        degv = dg_r[...][0, :, :1]
        norm = 1.0 / jnp.maximum(degv, 1.0)
        c1 = c1_r[...]
        lm = jnp.where(degv > 0, _dot(c1, lw_r[...]), _dot(c1, ew_r[...]))
        cur2 = _rrelu(ag_r[...][0] * norm + lm)
        ch = _l2norm(cur2)
        hv = h_r[...]
        tw = jax.nn.sigmoid(_dot(hv, tgw_r[...]) + tgb_r[...][None, :])
        hn = tw * ch + (1.0 - tw) * hv
        hn_r[...] = hn
        hw_r[...] = _dot(hn, w0_r[...])

    return pl.pallas_call(
        body,
        grid=(ROWS // BLK,),
        in_specs=[pl.BlockSpec((1, BLK, H), _map_sp),
                  pl.BlockSpec((1, BLK, H), _map_sp),
                  pl.BlockSpec((BLK, H), _map_row),
                  pl.BlockSpec((BLK, H), _map_row),
                  pl.BlockSpec((H, H), _map0),
                  pl.BlockSpec((H, H), _map0),
                  pl.BlockSpec((H, H), _map0),
                  pl.BlockSpec((H,), lambda i: (0,)),
                  pl.BlockSpec((H, H), _map0)],
        out_specs=[pl.BlockSpec((BLK, H), _map_row),
                   pl.BlockSpec((BLK, H), _map_row)],
        out_shape=[jax.ShapeDtypeStruct((ROWS, H), jnp.float32),
                   jax.ShapeDtypeStruct((ROWS, H), jnp.float32)],
    )(agg, deg, cur1, h, lw, ew, tgw, tgb, w0)


def _tc_gru(sumr, cnt, emb_rel, h0, w_ih, b_ih, w_hh, b_hh):
    def body(sr_r, ct_r, er_r, h0_r, wih_r, bih_r, whh_r, bhh_r, out_r):
        ctv = ct_r[...]
        c = 2.0 * (ctv[0, :, :1] + ctv[1, :, :1])
        srv = sr_r[...]
        sr = srv[0] + srv[1]
        x_in = jnp.where(c > 0, sr / jnp.maximum(c, 1.0), 0.0)
        er = er_r[...]
        wih = wih_r[...]
        gi = (_dot_t(er, wih[:, :H]) + _dot_t(x_in, wih[:, H:])
              + bih_r[...][None, :])
        h0v = h0_r[...]
        gh = _dot_t(h0v, whh_r[...]) + bhh_r[...][None, :]
        r = jax.nn.sigmoid(gi[:, :H] + gh[:, :H])
        z = jax.nn.sigmoid(gi[:, H:2 * H] + gh[:, H:2 * H])
        n = jnp.tanh(gi[:, 2 * H:] + r * gh[:, 2 * H:])
        out_r[...] = _l2norm((1.0 - z) * n + z * h0v)

    return pl.pallas_call(
        body,
        grid=(1,),
        in_specs=[pl.BlockSpec((NC, RELS, H), lambda i: (0, 0, 0)),
                  pl.BlockSpec((NC, RELS, H), lambda i: (0, 0, 0)),
                  pl.BlockSpec((RELS, H), _map0),
                  pl.BlockSpec((RELS, H), _map0),
                  pl.BlockSpec((3 * H, 2 * H), _map0),
                  pl.BlockSpec((3 * H,), lambda i: (0,)),
                  pl.BlockSpec((3 * H, H), _map0),
                  pl.BlockSpec((3 * H,), lambda i: (0,))],
        out_specs=pl.BlockSpec((RELS, H), _map0),
        out_shape=jax.ShapeDtypeStruct((RELS, H), jnp.float32),
    )(sumr, cnt, emb_rel, h0, w_ih, b_ih, w_hh, b_hh)


def _sc_mesh():
    return plsc.VectorSubcoreMesh(core_axis_name="c", subcore_axis_name="s")


def _zero_span(s, zbuf, tab_sh, total_rows):
    per = total_rows // NS
    base = s * per
    off = 0
    while off < per:
        n = min(CHUNK, per - off)
        pltpu.sync_copy(zbuf.at[pl.ds(0, n)], tab_sh.at[pl.ds(base + off, n)])
        off += n


def _stream_job(tab_h, gidx, tgt_sh, sidx, rows0, rows1, sem0, sem1,
                nch=CH, base=0):
    @pl.loop(0, nch, step=2)
    def _(i):
        j = base + i
        g0 = pltpu.async_copy(tab_h.at[gidx.at[j]], rows0, sem0)
        g1 = pltpu.async_copy(tab_h.at[gidx.at[j + 1]], rows1, sem1)
        g0.wait()
        pltpu.sync_copy(rows0, tgt_sh.at[sidx.at[j]], add=True)
        g1.wait()
        pltpu.sync_copy(rows1, tgt_sh.at[sidx.at[j + 1]], add=True)


def _ones_job(ones, tgt_sh, sidx, nch=CH, base=0):
    @pl.loop(0, nch)
    def _(i):
        pltpu.sync_copy(ones, tgt_sh.at[sidx.at[base + i]], add=True)


def _copy_out(sh, out_ref, s, total_rows):
    per = total_rows // NS
    pltpu.sync_copy(sh.at[pl.ds(s * per, per)],
                    out_ref.at[pl.ds(s * per, per)])


def _sc_stage_h(h, src_i, dst_i, et_i, dstl_i, zeros_h, ones_h):
    out_type = [
        jax.ShapeDtypeStruct((NC, HALF, H), jnp.float32),
        jax.ShapeDtypeStruct((NC, RELS, H), jnp.float32),
        jax.ShapeDtypeStruct((NC, RELS, H), jnp.float32),
    ]
    scratch = [
        pltpu.VMEM((CH, CHUNK), jnp.int32),
        pltpu.VMEM((CH, CHUNK), jnp.int32),
        pltpu.VMEM((CHUNK, H), jnp.float32),
        pltpu.VMEM((CHUNK, H), jnp.float32),
        pltpu.VMEM_SHARED((RPC, H), jnp.float32),
        pltpu.VMEM_SHARED((RELS, H), jnp.float32),
        pltpu.VMEM_SHARED((RELS, H), jnp.float32),
        pltpu.SemaphoreType.DMA,
        pltpu.SemaphoreType.DMA,
    ]

    @functools.partial(pl.kernel, out_type=out_type, mesh=_sc_mesh(),
                       scratch_types=scratch)
    def k(h_h, src_h, dst_h, et_h, dstl_h, z_h, o_h,
          deg_o, cnt_o, sumr_o,
          idx_s, et_s, rows0, rows1,
          deg_sh, cnt_sh, sumr_sh, sem0, sem1):
        c = lax.axis_index("c")
        s = lax.axis_index("s")
        pltpu.sync_copy(z_h, rows0)
        _zero_span(s, rows0, deg_sh, RPC)
        _zero_span(s, rows0, cnt_sh, RELS)
        _zero_span(s, rows0, sumr_sh, RELS)
        pltpu.sync_copy(et_h.at[s], et_s)
        plsc.subcore_barrier()
        base = c * HCH
        pltpu.sync_copy(src_h.at[s], idx_s)
        _stream_job(h_h, idx_s, sumr_sh, et_s, rows0, rows1, sem0, sem1,
                    nch=HCH, base=base)
        pltpu.sync_copy(dst_h.at[s], idx_s)
        _stream_job(h_h, idx_s, sumr_sh, et_s, rows0, rows1, sem0, sem1,
                    nch=HCH, base=base)
        pltpu.sync_copy(o_h, rows1)
        pltpu.sync_copy(dstl_h.at[c].at[s], idx_s)
        _ones_job(rows1, deg_sh, idx_s)
        _ones_job(rows1, cnt_sh, et_s, nch=HCH, base=base)
        plsc.subcore_barrier()
        _copy_out(deg_sh, deg_o.at[c], s, HALF)
        _copy_out(cnt_sh, cnt_o.at[c], s, RELS)
        _copy_out(sumr_sh, sumr_o.at[c], s, RELS)

    return k(h, src_i, dst_i, et_i, dstl_i, zeros_h, ones_h)


def _sc_agg(hw, erw, src_i, et_i, dstl_i, zeros_h, tok):
    out_type = jax.ShapeDtypeStruct((NC, HALF, H), jnp.float32)
    scratch = [
        pltpu.VMEM((CH, CHUNK), jnp.int32),
        pltpu.VMEM((CH, CHUNK), jnp.int32),
        pltpu.VMEM((CHUNK, H), jnp.float32),
        pltpu.VMEM((CHUNK, H), jnp.float32),
        pltpu.VMEM_SHARED((RPC, H), jnp.float32),
        pltpu.SemaphoreType.DMA,
        pltpu.SemaphoreType.DMA,
    ]

    @functools.partial(pl.kernel, out_type=out_type, mesh=_sc_mesh(),
                       scratch_types=scratch)
    def k(hw_h, erw_h, src_h, et_h, dstl_h, z_h, tok_h,
          agg_o,
          idx_s, dstl_s, rows0, rows1,
          agg_sh, sem0, sem1):
        c = lax.axis_index("c")
        s = lax.axis_index("s")
        pltpu.sync_copy(z_h, rows0)
        _zero_span(s, rows0, agg_sh, RPC)
        pltpu.sync_copy(dstl_h.at[c].at[s], dstl_s)
        plsc.subcore_barrier()
        pltpu.sync_copy(src_h.at[s], idx_s)
        _stream_job(hw_h, idx_s, agg_sh, dstl_s, rows0, rows1, sem0, sem1)
        pltpu.sync_copy(et_h.at[s], idx_s)
        _stream_job(erw_h, idx_s, agg_sh, dstl_s, rows0, rows1, sem0, sem1)
        plsc.subcore_barrier()
        _copy_out(agg_sh, agg_o.at[c], s, HALF)

    return k(hw, erw, src_i, et_i, dstl_i, zeros_h, tok)


def kernel(edge_src, edge_dst, edge_type, dynamic_emb, emb_rel, w_ih, b_ih,
           w_hh, b_hh, time_gate_w, time_gate_b, w_neigh_0, loop_w_0,
           evolve_w_0, w_neigh_1, loop_w_1, evolve_w_1):
    pad = EPAD - E
    srcp = jnp.pad(edge_src, ((0, 0), (0, pad))).reshape(T, NS, CH, CHUNK)
    dstp = jnp.pad(edge_dst, ((0, 0), (0, pad)),
                   constant_values=NUM_ENTS).reshape(T, NS, CH, CHUNK)
    etp = jnp.pad(edge_type, ((0, 0), (0, pad)),
                  constant_values=2 * NUM_RELS).reshape(T, NS, CH, CHUNK)
    dstl = jnp.stack(
        [jnp.where((dstp >= c * HALF) & (dstp < (c + 1) * HALF),
                   dstp - c * HALF, JUNK) for c in range(NC)], axis=1)
    zeros_h = jnp.zeros((CHUNK, H), jnp.float32)
    ones_h = jnp.ones((CHUNK, H), jnp.float32)

    embp = jnp.pad(dynamic_emb, ((0, ROWS - NUM_ENTS), (0, 0)))
    erp = jnp.pad(emb_rel, ((0, RELS - 2 * NUM_RELS), (0, 0)))

    h, hw1, erw1, erw2 = _tc_prep(embp, w_neigh_0, w_neigh_1, erp)
    h0 = erp
    hist = []
    for t in range(T):
        deg, cnt, sumr = _sc_stage_h(
            h, srcp[t], dstp[t], etp[t], dstl[t], zeros_h, ones_h)
        tok1 = deg[0, :8]
        agg1 = _sc_agg(hw1, erw1, srcp[t], etp[t], dstl[t], zeros_h, tok1)
        cur1, hw2 = _tc_mid(agg1, deg, h, loop_w_0, evolve_w_0, w_neigh_1)
        h0 = _tc_gru(sumr, cnt, erp, h0, w_ih, b_ih, w_hh, b_hh)
        tok2 = agg1[0, :8]
        agg2 = _sc_agg(hw2, erw2, srcp[t], etp[t], dstl[t], zeros_h, tok2)
        h, hw1 = _tc_post(agg2, deg, cur1, h, loop_w_1, evolve_w_1,
                          time_gate_w, time_gate_b, w_neigh_0)
        hist.append(h[:NUM_ENTS])
    return jnp.stack(hist, axis=0), h0[:2 * NUM_RELS]


# speedup vs baseline: 1.0553x; 1.0014x over previous
import functools

import jax
import jax.numpy as jnp
from jax import lax
from jax.experimental import pallas as pl
from jax.experimental.pallas import tpu as pltpu
from jax.experimental.pallas import tpu_sc as plsc

NUM_ENTS = 10000
NUM_RELS = 200
H = 128
T = 3
E = 320000
SLOPE = (1.0 / 8.0 + 1.0 / 3.0) / 2.0

NC = 2
NS = 16
CHUNK = 128
CH = 160
HCH = CH // 2
EPW = CH * CHUNK
EPAD = EPW * NS
ROWS = 10240
HALF = ROWS // 2
RPC = 5248
JUNK = HALF
RELS = 512
BLK = 2560


def _l2norm(x):
    n = jnp.sqrt(jnp.sum(x * x, axis=-1, keepdims=True))
    return x / jnp.maximum(n, 1e-12)


def _rrelu(x):
    return jnp.where(x >= 0, x, x * SLOPE)


def _dot(a, b):
    return lax.dot_general(a, b, (((1,), (0,)), ((), ())),
                           preferred_element_type=jnp.float32)


def _dot_t(a, b):
    return lax.dot_general(a, b, (((1,), (1,)), ((), ())),
                           preferred_element_type=jnp.float32)


def _map_sp(i):
    return (i // 2, i % 2, 0)


def _map_row(i):
    return (i, 0)


def _map0(i):
    return (0, 0)


def _tc_prep(emb, w0, w1, emb_rel):
    def body(emb_r, w0_r, w1_r, er_r, h_r, hw_r, e1_r, e2_r):
        h = _l2norm(emb_r[...])
        h_r[...] = h
        hw_r[...] = _dot(h, w0_r[...])
        er = er_r[...]
        e1_r[...] = _dot(er, w0_r[...])
        e2_r[...] = _dot(er, w1_r[...])

    return pl.pallas_call(
        body,
        grid=(ROWS // BLK,),
        in_specs=[pl.BlockSpec((BLK, H), _map_row),
                  pl.BlockSpec((H, H), _map0),
                  pl.BlockSpec((H, H), _map0),
                  pl.BlockSpec((RELS, H), _map0)],
        out_specs=[pl.BlockSpec((BLK, H), _map_row),
                   pl.BlockSpec((BLK, H), _map_row),
                   pl.BlockSpec((RELS, H), _map0),
                   pl.BlockSpec((RELS, H), _map0)],
        out_shape=[jax.ShapeDtypeStruct((ROWS, H), jnp.float32),
                   jax.ShapeDtypeStruct((ROWS, H), jnp.float32),
                   jax.ShapeDtypeStruct((RELS, H), jnp.float32),
                   jax.ShapeDtypeStruct((RELS, H), jnp.float32)],
    )(emb, w0, w1, emb_rel)


def _tc_mid(agg, deg, h, lw, ew, w_next):
    def body(ag_r, dg_r, h_r, lw_r, ew_r, wn_r, cur_r, hw_r):
        degv = dg_r[...][0, :, :1]
        norm = 1.0 / jnp.maximum(degv, 1.0)
        hv = h_r[...]
        lm = jnp.where(degv > 0, _dot(hv, lw_r[...]), _dot(hv, ew_r[...]))
        cur = _rrelu(ag_r[...][0] * norm + lm)
        cur_r[...] = cur
        hw_r[...] = _dot(cur, wn_r[...])

    return pl.pallas_call(
        body,
        grid=(ROWS // BLK,),
        in_specs=[pl.BlockSpec((1, BLK, H), _map_sp),
                  pl.BlockSpec((1, BLK, H), _map_sp),
                  pl.BlockSpec((BLK, H), _map_row),
                  pl.BlockSpec((H, H), _map0),
                  pl.BlockSpec((H, H), _map0),
                  pl.BlockSpec((H, H), _map0)],
        out_specs=[pl.BlockSpec((BLK, H), _map_row),
                   pl.BlockSpec((BLK, H), _map_row)],
        out_shape=[jax.ShapeDtypeStruct((ROWS, H), jnp.float32),
                   jax.ShapeDtypeStruct((ROWS, H), jnp.float32)],
    )(agg, deg, h, lw, ew, w_next)


def _tc_post(agg, deg, cur1, h, lw, ew, tgw, tgb, w0):
    def body(ag_r, dg_r, c1_r, h_r, lw_r, ew_r, tgw_r, tgb_r, w0_r,
             hn_r, hw_r):
        degv = dg_r[...][0, :, :1]
        norm = 1.0 / jnp.maximum(degv, 1.0)
        c1 = c1_r[...]
        lm = jnp.where(degv > 0, _dot(c1, lw_r[...]), _dot(c1, ew_r[...]))
        cur2 = _rrelu(ag_r[...][0] * norm + lm)
        ch = _l2norm(cur2)
        hv = h_r[...]
        tw = jax.nn.sigmoid(_dot(hv, tgw_r[...]) + tgb_r[...][None, :])
        hn = tw * ch + (1.0 - tw) * hv
        hn_r[...] = hn
        hw_r[...] = _dot(hn, w0_r[...])

    return pl.pallas_call(
        body,
        grid=(ROWS // BLK,),
        in_specs=[pl.BlockSpec((1, BLK, H), _map_sp),
                  pl.BlockSpec((1, BLK, H), _map_sp),
                  pl.BlockSpec((BLK, H), _map_row),
                  pl.BlockSpec((BLK, H), _map_row),
                  pl.BlockSpec((H, H), _map0),
                  pl.BlockSpec((H, H), _map0),
                  pl.BlockSpec((H, H), _map0),
                  pl.BlockSpec((H,), lambda i: (0,)),
                  pl.BlockSpec((H, H), _map0)],
        out_specs=[pl.BlockSpec((BLK, H), _map_row),
                   pl.BlockSpec((BLK, H), _map_row)],
        out_shape=[jax.ShapeDtypeStruct((ROWS, H), jnp.float32),
                   jax.ShapeDtypeStruct((ROWS, H), jnp.float32)],
    )(agg, deg, cur1, h, lw, ew, tgw, tgb, w0)


def _tc_gru(sumr, cnt, emb_rel, h0, w_ih, b_ih, w_hh, b_hh):
    def body(sr_r, ct_r, er_r, h0_r, wih_r, bih_r, whh_r, bhh_r, out_r):
        ctv = ct_r[...]
        c = 2.0 * (ctv[0, :, :1] + ctv[1, :, :1])
        srv = sr_r[...]
        sr = srv[0] + srv[1]
        x_in = jnp.where(c > 0, sr / jnp.maximum(c, 1.0), 0.0)
        er = er_r[...]
        wih = wih_r[...]
        gi = (_dot_t(er, wih[:, :H]) + _dot_t(x_in, wih[:, H:])
              + bih_r[...][None, :])
        h0v = h0_r[...]
        gh = _dot_t(h0v, whh_r[...]) + bhh_r[...][None, :]
        r = jax.nn.sigmoid(gi[:, :H] + gh[:, :H])
        z = jax.nn.sigmoid(gi[:, H:2 * H] + gh[:, H:2 * H])
        n = jnp.tanh(gi[:, 2 * H:] + r * gh[:, 2 * H:])
        out_r[...] = _l2norm((1.0 - z) * n + z * h0v)

    return pl.pallas_call(
        body,
        grid=(1,),
        in_specs=[pl.BlockSpec((NC, RELS, H), lambda i: (0, 0, 0)),
                  pl.BlockSpec((NC, RELS, H), lambda i: (0, 0, 0)),
                  pl.BlockSpec((RELS, H), _map0),
                  pl.BlockSpec((RELS, H), _map0),
                  pl.BlockSpec((3 * H, 2 * H), _map0),
                  pl.BlockSpec((3 * H,), lambda i: (0,)),
                  pl.BlockSpec((3 * H, H), _map0),
                  pl.BlockSpec((3 * H,), lambda i: (0,))],
        out_specs=pl.BlockSpec((RELS, H), _map0),
        out_shape=jax.ShapeDtypeStruct((RELS, H), jnp.float32),
    )(sumr, cnt, emb_rel, h0, w_ih, b_ih, w_hh, b_hh)


def _tc_tok(x):
    def body(x_r, o_r):
        o_r[...] = x_r[...] * 1.0

    return pl.pallas_call(
        body,
        grid=(1,),
        in_specs=[pl.BlockSpec((8, H), _map0)],
        out_specs=pl.BlockSpec((8, H), _map0),
        out_shape=jax.ShapeDtypeStruct((8, H), jnp.float32),
    )(x)


def _sc_mesh():
    return plsc.VectorSubcoreMesh(core_axis_name="c", subcore_axis_name="s")


def _zero_span(s, zbuf, tab_sh, total_rows):
    per = total_rows // NS
    base = s * per
    off = 0
    while off < per:
        n = min(CHUNK, per - off)
        pltpu.sync_copy(zbuf.at[pl.ds(0, n)], tab_sh.at[pl.ds(base + off, n)])
        off += n


NBUF = 2


def _stream_job(tab_h, gidx, tgt_sh, sidx, rows, gsems, ssems,
                nch=CH, base=0):
    @pl.loop(0, nch, step=NBUF)
    def _(i):
        j = base + i
        gs = [pltpu.async_copy(tab_h.at[gidx.at[j + k]], rows[k], gsems[k])
              for k in range(NBUF)]
        ss = []
        for k in range(NBUF):
            gs[k].wait()
            ss.append(pltpu.async_copy(rows[k], tgt_sh.at[sidx.at[j + k]],
                                       ssems[k], add=True))
        for s_ in ss:
            s_.wait()


def _ones_job(ones, tgt_sh, sidx, ssems, nch=CH, base=0):
    @pl.loop(0, nch, step=NBUF)
    def _(i):
        j = base + i
        ss = [pltpu.async_copy(ones, tgt_sh.at[sidx.at[j + k]], ssems[k],
                               add=True)
              for k in range(NBUF)]
        for s_ in ss:
            s_.wait()


def _copy_out(sh, out_ref, s, total_rows):
    per = total_rows // NS
    pltpu.sync_copy(sh.at[pl.ds(s * per, per)],
                    out_ref.at[pl.ds(s * per, per)])


def _sc_stage_h(h, src_i, dst_i, et_i, dstl_i, zeros_h, ones_h):
    out_type = [
        jax.ShapeDtypeStruct((NC, HALF, H), jnp.float32),
        jax.ShapeDtypeStruct((NC, RELS, H), jnp.float32),
        jax.ShapeDtypeStruct((NC, RELS, H), jnp.float32),
    ]
    scratch = [
        pltpu.VMEM((CH, CHUNK), jnp.int32),
        pltpu.VMEM((CH, CHUNK), jnp.int32),
    ] + [pltpu.VMEM((CHUNK, H), jnp.float32)] * NBUF + [
        pltpu.VMEM_SHARED((RPC, H), jnp.float32),
        pltpu.VMEM_SHARED((RELS, H), jnp.float32),
        pltpu.VMEM_SHARED((RELS, H), jnp.float32),
    ] + [pltpu.SemaphoreType.DMA] * (2 * NBUF)

    @functools.partial(pl.kernel, out_type=out_type, mesh=_sc_mesh(),
                       scratch_types=scratch)
    def k(h_h, src_h, dst_h, et_h, dstl_h, z_h, o_h,
          deg_o, cnt_o, sumr_o,
          idx_s, et_s, *bufs):
        rows = list(bufs[:NBUF])
        deg_sh, cnt_sh, sumr_sh = bufs[NBUF:NBUF + 3]
        gsems = list(bufs[NBUF + 3:NBUF + 3 + NBUF])
        ssems = list(bufs[NBUF + 3 + NBUF:])
        c = lax.axis_index("c")
        s = lax.axis_index("s")
        pltpu.sync_copy(z_h, rows[0])
        _zero_span(s, rows[0], deg_sh, RPC)
        _zero_span(s, rows[0], cnt_sh, RELS)
        _zero_span(s, rows[0], sumr_sh, RELS)
        pltpu.sync_copy(et_h.at[s], et_s)
        plsc.subcore_barrier()
        base = c * HCH

        def sync_stream(tab_h, gidx, tgt_sh, sidx, nch, b0):
            @pl.loop(0, nch, step=2)
            def _(i):
                j = b0 + i
                g0 = pltpu.async_copy(tab_h.at[gidx.at[j]], rows[0],
                                      gsems[0])
                g1 = pltpu.async_copy(tab_h.at[gidx.at[j + 1]], rows[1],
                                      gsems[1])
                g0.wait()
                pltpu.sync_copy(rows[0], tgt_sh.at[sidx.at[j]], add=True)
                g1.wait()
                pltpu.sync_copy(rows[1], tgt_sh.at[sidx.at[j + 1]], add=True)

        pltpu.sync_copy(src_h.at[s], idx_s)
        sync_stream(h_h, idx_s, sumr_sh, et_s, HCH, base)
        pltpu.sync_copy(dst_h.at[s], idx_s)
        sync_stream(h_h, idx_s, sumr_sh, et_s, HCH, base)
        pltpu.sync_copy(o_h, rows[0])
        pltpu.sync_copy(dstl_h.at[c].at[s], idx_s)

        @pl.loop(0, CH)
        def _(i):
            pltpu.sync_copy(rows[0], deg_sh.at[idx_s.at[i]], add=True)

        @pl.loop(0, HCH)
        def _(i):
            pltpu.sync_copy(rows[0], cnt_sh.at[et_s.at[base + i]], add=True)

        plsc.subcore_barrier()
        _copy_out(deg_sh, deg_o.at[c], s, HALF)
        _copy_out(cnt_sh, cnt_o.at[c], s, RELS)
        _copy_out(sumr_sh, sumr_o.at[c], s, RELS)

    return k(h, src_i, dst_i, et_i, dstl_i, zeros_h, ones_h)


def _sc_agg(hw, erw, src_i, et_i, dstl_i, zeros_h, tok):
    out_type = jax.ShapeDtypeStruct((NC, HALF, H), jnp.float32)
    scratch = [
        pltpu.VMEM((CH, CHUNK), jnp.int32),
        pltpu.VMEM((CH, CHUNK), jnp.int32),
    ] + [pltpu.VMEM((CHUNK, H), jnp.float32)] * NBUF + [
        pltpu.VMEM_SHARED((RPC, H), jnp.float32),
    ] + [pltpu.SemaphoreType.DMA] * (2 * NBUF)

    @functools.partial(pl.kernel, out_type=out_type, mesh=_sc_mesh(),
                       scratch_types=scratch)
    def k(hw_h, erw_h, src_h, et_h, dstl_h, z_h, tok_h,
          agg_o,
          idx_s, dstl_s, *bufs):
        rows = list(bufs[:NBUF])
        agg_sh = bufs[NBUF]
        gsems = list(bufs[NBUF + 1:NBUF + 1 + NBUF])
        ssems = list(bufs[NBUF + 1 + NBUF:])
        c = lax.axis_index("c")
        s = lax.axis_index("s")
        pltpu.sync_copy(z_h, rows[0])
        _zero_span(s, rows[0], agg_sh, RPC)
        pltpu.sync_copy(dstl_h.at[c].at[s], dstl_s)
        plsc.subcore_barrier()
        pltpu.sync_copy(src_h.at[s], idx_s)
        _stream_job(hw_h, idx_s, agg_sh, dstl_s, rows, gsems, ssems)
        pltpu.sync_copy(et_h.at[s], idx_s)
        _stream_job(erw_h, idx_s, agg_sh, dstl_s, rows, gsems, ssems)
        plsc.subcore_barrier()
        _copy_out(agg_sh, agg_o.at[c], s, HALF)

    return k(hw, erw, src_i, et_i, dstl_i, zeros_h, tok)


def kernel(edge_src, edge_dst, edge_type, dynamic_emb, emb_rel, w_ih, b_ih,
           w_hh, b_hh, time_gate_w, time_gate_b, w_neigh_0, loop_w_0,
           evolve_w_0, w_neigh_1, loop_w_1, evolve_w_1):
    pad = EPAD - E
    srcp = jnp.pad(edge_src, ((0, 0), (0, pad))).reshape(T, NS, CH, CHUNK)
    dstp = jnp.pad(edge_dst, ((0, 0), (0, pad)),
                   constant_values=NUM_ENTS).reshape(T, NS, CH, CHUNK)
    etp = jnp.pad(edge_type, ((0, 0), (0, pad)),
                  constant_values=2 * NUM_RELS).reshape(T, NS, CH, CHUNK)
    dstl = jnp.stack(
        [jnp.where((dstp >= c * HALF) & (dstp < (c + 1) * HALF),
                   dstp - c * HALF, JUNK) for c in range(NC)], axis=1)
    zeros_h = jnp.zeros((CHUNK, H), jnp.float32)
    ones_h = jnp.ones((CHUNK, H), jnp.float32)

    embp = jnp.pad(dynamic_emb, ((0, ROWS - NUM_ENTS), (0, 0)))
    erp = jnp.pad(emb_rel, ((0, RELS - 2 * NUM_RELS), (0, 0)))

    h, hw1, erw1, erw2 = _tc_prep(embp, w_neigh_0, w_neigh_1, erp)
    h0 = erp
    hist = []
    for t in range(T):
        deg, cnt, sumr = _sc_stage_h(
            h, srcp[t], dstp[t], etp[t], dstl[t], zeros_h, ones_h)
        tok1 = _tc_tok(deg[0, :8])
        agg1 = _sc_agg(hw1, erw1, srcp[t], etp[t], dstl[t], zeros_h, tok1)
        cur1, hw2 = _tc_mid(agg1, deg, h, loop_w_0, evolve_w_0, w_neigh_1)
        h0 = _tc_gru(sumr, cnt, erp, h0, w_ih, b_ih, w_hh, b_hh)
        tok2 = _tc_tok(agg1[0, :8])
        agg2 = _sc_agg(hw2, erw2, srcp[t], etp[t], dstl[t], zeros_h, tok2)
        h, hw1 = _tc_post(agg2, deg, cur1, h, loop_w_1, evolve_w_1,
                          time_gate_w, time_gate_b, w_neigh_0)
        hist.append(h[:NUM_ENTS])
    return jnp.stack(hist, axis=0), h0[:2 * NUM_RELS]


# speedup vs baseline: 1.0584x; 1.0030x over previous
import functools

import jax
import jax.numpy as jnp
from jax import lax
from jax.experimental import pallas as pl
from jax.experimental.pallas import tpu as pltpu
from jax.experimental.pallas import tpu_sc as plsc

NUM_ENTS = 10000
NUM_RELS = 200
H = 128
T = 3
E = 320000
SLOPE = (1.0 / 8.0 + 1.0 / 3.0) / 2.0

NC = 2
NS = 16
CHUNK = 128
CH = 160
HCH = CH // 2
EPW = CH * CHUNK
EPAD = EPW * NS
ROWS = 10240
HALF = ROWS // 2
RPC = 5248
JUNK = HALF
RELS = 512
BLK = 2560


def _l2norm(x):
    n = jnp.sqrt(jnp.sum(x * x, axis=-1, keepdims=True))
    return x / jnp.maximum(n, 1e-12)


def _rrelu(x):
    return jnp.where(x >= 0, x, x * SLOPE)


def _dot(a, b):
    return lax.dot_general(a, b, (((1,), (0,)), ((), ())),
                           preferred_element_type=jnp.float32)


def _dot_t(a, b):
    return lax.dot_general(a, b, (((1,), (1,)), ((), ())),
                           preferred_element_type=jnp.float32)


def _map_sp(i):
    return (i // 2, i % 2, 0)


def _map_row(i):
    return (i, 0)


def _map0(i):
    return (0, 0)


def _tc_prep(emb, w0, w1, emb_rel):
    def body(emb_r, w0_r, w1_r, er_r, h_r, hw_r, e1_r, e2_r):
        h = _l2norm(emb_r[...])
        h_r[...] = h
        hw_r[...] = _dot(h, w0_r[...])
        er = er_r[...]
        e1_r[...] = _dot(er, w0_r[...])
        e2_r[...] = _dot(er, w1_r[...])

    return pl.pallas_call(
        body,
        grid=(ROWS // BLK,),
        in_specs=[pl.BlockSpec((BLK, H), _map_row),
                  pl.BlockSpec((H, H), _map0),
                  pl.BlockSpec((H, H), _map0),
                  pl.BlockSpec((RELS, H), _map0)],
        out_specs=[pl.BlockSpec((BLK, H), _map_row),
                   pl.BlockSpec((BLK, H), _map_row),
                   pl.BlockSpec((RELS, H), _map0),
                   pl.BlockSpec((RELS, H), _map0)],
        out_shape=[jax.ShapeDtypeStruct((ROWS, H), jnp.float32),
                   jax.ShapeDtypeStruct((ROWS, H), jnp.float32),
                   jax.ShapeDtypeStruct((RELS, H), jnp.float32),
                   jax.ShapeDtypeStruct((RELS, H), jnp.float32)],
    )(emb, w0, w1, emb_rel)


def _tc_mid(agg, deg, h, lw, ew, w_next):
    def body(ag_r, dg_r, h_r, lw_r, ew_r, wn_r, cur_r, hw_r):
        degv = dg_r[...][0, :, :1]
        norm = 1.0 / jnp.maximum(degv, 1.0)
        hv = h_r[...]
        lm = jnp.where(degv > 0, _dot(hv, lw_r[...]), _dot(hv, ew_r[...]))
        cur = _rrelu(ag_r[...][0] * norm + lm)
        cur_r[...] = cur
        hw_r[...] = _dot(cur, wn_r[...])

    return pl.pallas_call(
        body,
        grid=(ROWS // BLK,),
        in_specs=[pl.BlockSpec((1, BLK, H), _map_sp),
                  pl.BlockSpec((1, BLK, H), _map_sp),
                  pl.BlockSpec((BLK, H), _map_row),
                  pl.BlockSpec((H, H), _map0),
                  pl.BlockSpec((H, H), _map0),
                  pl.BlockSpec((H, H), _map0)],
        out_specs=[pl.BlockSpec((BLK, H), _map_row),
                   pl.BlockSpec((BLK, H), _map_row)],
        out_shape=[jax.ShapeDtypeStruct((ROWS, H), jnp.float32),
                   jax.ShapeDtypeStruct((ROWS, H), jnp.float32)],
    )(agg, deg, h, lw, ew, w_next)


def _tc_post(agg, deg, cur1, h, lw, ew, tgw, tgb, w0):
    def body(ag_r, dg_r, c1_r, h_r, lw_r, ew_r, tgw_r, tgb_r, w0_r,
             hn_r, hw_r):
        degv = dg_r[...][0, :, :1]
        norm = 1.0 / jnp.maximum(degv, 1.0)
        c1 = c1_r[...]
        lm = jnp.where(degv > 0, _dot(c1, lw_r[...]), _dot(c1, ew_r[...]))
        cur2 = _rrelu(ag_r[...][0] * norm + lm)
        ch = _l2norm(cur2)
        hv = h_r[...]
        tw = jax.nn.sigmoid(_dot(hv, tgw_r[...]) + tgb_r[...][None, :])
        hn = tw * ch + (1.0 - tw) * hv
        hn_r[...] = hn
        hw_r[...] = _dot(hn, w0_r[...])

    return pl.pallas_call(
        body,
        grid=(ROWS // BLK,),
        in_specs=[pl.BlockSpec((1, BLK, H), _map_sp),
                  pl.BlockSpec((1, BLK, H), _map_sp),
                  pl.BlockSpec((BLK, H), _map_row),
                  pl.BlockSpec((BLK, H), _map_row),
                  pl.BlockSpec((H, H), _map0),
                  pl.BlockSpec((H, H), _map0),
                  pl.BlockSpec((H, H), _map0),
                  pl.BlockSpec((H,), lambda i: (0,)),
                  pl.BlockSpec((H, H), _map0)],
        out_specs=[pl.BlockSpec((BLK, H), _map_row),
                   pl.BlockSpec((BLK, H), _map_row)],
        out_shape=[jax.ShapeDtypeStruct((ROWS, H), jnp.float32),
                   jax.ShapeDtypeStruct((ROWS, H), jnp.float32)],
    )(agg, deg, cur1, h, lw, ew, tgw, tgb, w0)


def _tc_gru(sumr, cnt, emb_rel, h0, w_ih, b_ih, w_hh, b_hh):
    def body(sr_r, ct_r, er_r, h0_r, wih_r, bih_r, whh_r, bhh_r, out_r):
        ctv = ct_r[...]
        c = 2.0 * (ctv[0, :, :1] + ctv[1, :, :1])
        srv = sr_r[...]
        sr = srv[0] + srv[1]
        x_in = jnp.where(c > 0, sr / jnp.maximum(c, 1.0), 0.0)
        er = er_r[...]
        wih = wih_r[...]
        gi = (_dot_t(er, wih[:, :H]) + _dot_t(x_in, wih[:, H:])
              + bih_r[...][None, :])
        h0v = h0_r[...]
        gh = _dot_t(h0v, whh_r[...]) + bhh_r[...][None, :]
        r = jax.nn.sigmoid(gi[:, :H] + gh[:, :H])
        z = jax.nn.sigmoid(gi[:, H:2 * H] + gh[:, H:2 * H])
        n = jnp.tanh(gi[:, 2 * H:] + r * gh[:, 2 * H:])
        out_r[...] = _l2norm((1.0 - z) * n + z * h0v)

    return pl.pallas_call(
        body,
        grid=(1,),
        in_specs=[pl.BlockSpec((NC, RELS, H), lambda i: (0, 0, 0)),
                  pl.BlockSpec((NC, RELS, H), lambda i: (0, 0, 0)),
                  pl.BlockSpec((RELS, H), _map0),
                  pl.BlockSpec((RELS, H), _map0),
                  pl.BlockSpec((3 * H, 2 * H), _map0),
                  pl.BlockSpec((3 * H,), lambda i: (0,)),
                  pl.BlockSpec((3 * H, H), _map0),
                  pl.BlockSpec((3 * H,), lambda i: (0,))],
        out_specs=pl.BlockSpec((RELS, H), _map0),
        out_shape=jax.ShapeDtypeStruct((RELS, H), jnp.float32),
    )(sumr, cnt, emb_rel, h0, w_ih, b_ih, w_hh, b_hh)


def _tc_tok(x):
    def body(x_r, o_r):
        o_r[...] = x_r[...] * 1.0

    return pl.pallas_call(
        body,
        grid=(1,),
        in_specs=[pl.BlockSpec((8, H), _map0)],
        out_specs=pl.BlockSpec((8, H), _map0),
        out_shape=jax.ShapeDtypeStruct((8, H), jnp.float32),
    )(x)


def _sc_mesh():
    return plsc.VectorSubcoreMesh(core_axis_name="c", subcore_axis_name="s")


def _zero_span(s, zbuf2d, tab_sh, total_rows):
    per = total_rows // NS
    base = s * per
    off = 0
    while off < per:
        n = min(CHUNK, per - off)
        pltpu.sync_copy(zbuf2d.at[pl.ds(0, n)],
                        tab_sh.at[pl.ds(base + off, n)])
        off += n


NBUF = 2
NG = 2


def _stream_job(tab_h, gidx, tgt_sh, sidx, rows, gsems, ssems,
                nch=CH, base=0):
    @pl.loop(0, nch, step=2)
    def _(i):
        j = base + i
        g0 = pltpu.async_copy(tab_h.at[gidx.at[j]], rows[0], gsems[0])
        g1 = pltpu.async_copy(tab_h.at[gidx.at[j + 1]], rows[1], gsems[1])
        g0.wait()
        s0 = pltpu.async_copy(rows[0], tgt_sh.at[sidx.at[j]], ssems[0],
                              add=True)
        g1.wait()
        s1 = pltpu.async_copy(rows[1], tgt_sh.at[sidx.at[j + 1]], ssems[1],
                              add=True)
        s0.wait()
        s1.wait()


def _reg_add(a, b):
    @pl.loop(0, CHUNK, step=2)
    def _(r):
        for rr in range(2):
            for q in range(8):
                sl = pl.ds(q * 16, 16)
                a[r + rr, sl] = a[r + rr, sl] + b[r + rr, sl]


def _merge_job(tabA, gidxA, tabB, gidxB, tgt_sh, sidx, rows, gsems, ssems,
               nch=CH, base=0):
    @pl.loop(0, nch, step=2)
    def _(i):
        j = base + i
        ga0 = pltpu.async_copy(tabA.at[gidxA.at[j]], rows[0], gsems[0])
        gb0 = pltpu.async_copy(tabB.at[gidxB.at[j]], rows[1], gsems[1])
        ga0.wait()
        gb0.wait()
        _reg_add(rows[0], rows[1])
        s0 = pltpu.async_copy(rows[0], tgt_sh.at[sidx.at[j]], ssems[0],
                              add=True)
        ga1 = pltpu.async_copy(tabA.at[gidxA.at[j + 1]], rows[2], gsems[0])
        gb1 = pltpu.async_copy(tabB.at[gidxB.at[j + 1]], rows[1], gsems[1])
        ga1.wait()
        gb1.wait()
        _reg_add(rows[2], rows[1])
        s1 = pltpu.async_copy(rows[2], tgt_sh.at[sidx.at[j + 1]], ssems[1],
                              add=True)
        s0.wait()
        s1.wait()


def _ones_job(ones, tgt_sh, sidx, ssems, nch=CH, base=0):
    @pl.loop(0, nch, step=NBUF)
    def _(i):
        j = base + i
        ss = [pltpu.async_copy(ones, tgt_sh.at[sidx.at[j + k]], ssems[k],
                               add=True)
              for k in range(NBUF)]
        for s_ in ss:
            s_.wait()


def _copy_out(sh, out_ref, s, total_rows):
    per = total_rows // NS
    pltpu.sync_copy(sh.at[pl.ds(s * per, per)],
                    out_ref.at[pl.ds(s * per, per)])


def _sc_stage_h(h, src_i, dst_i, et_i, dstl_i, zeros_h, ones_h):
    out_type = [
        jax.ShapeDtypeStruct((NC, HALF, H), jnp.float32),
        jax.ShapeDtypeStruct((NC, RELS, H), jnp.float32),
        jax.ShapeDtypeStruct((NC, RELS, H), jnp.float32),
    ]
    scratch = [
        pltpu.VMEM((CH, CHUNK), jnp.int32),
        pltpu.VMEM((CH, CHUNK), jnp.int32),
    ] + [pltpu.VMEM((CHUNK, H), jnp.float32)] * 2 + [
        pltpu.VMEM_SHARED((RPC, H), jnp.float32),
        pltpu.VMEM_SHARED((RELS, H), jnp.float32),
        pltpu.VMEM_SHARED((RELS, H), jnp.float32),
    ] + [pltpu.SemaphoreType.DMA] * 4

    @functools.partial(pl.kernel, out_type=out_type, mesh=_sc_mesh(),
                       scratch_types=scratch)
    def k(h_h, src_h, dst_h, et_h, dstl_h, z_h, o_h,
          deg_o, cnt_o, sumr_o,
          idx_s, et_s, *bufs):
        rows = list(bufs[:2])
        deg_sh, cnt_sh, sumr_sh = bufs[2:5]
        gsems = list(bufs[5:7])
        ssems = list(bufs[7:9])
        c = lax.axis_index("c")
        s = lax.axis_index("s")
        pltpu.sync_copy(z_h, rows[0])
        _zero_span(s, rows[0], deg_sh, RPC)
        _zero_span(s, rows[0], cnt_sh, RELS)
        _zero_span(s, rows[0], sumr_sh, RELS)
        pltpu.sync_copy(et_h.at[s], et_s)
        plsc.subcore_barrier()
        base = c * HCH
        pltpu.sync_copy(src_h.at[s], idx_s)
        _stream_job(h_h, idx_s, sumr_sh, et_s, rows, gsems, ssems,
                    nch=HCH, base=base)
        pltpu.sync_copy(dst_h.at[s], idx_s)
        _stream_job(h_h, idx_s, sumr_sh, et_s, rows, gsems, ssems,
                    nch=HCH, base=base)
        pltpu.sync_copy(o_h, rows[0])
        pltpu.sync_copy(dstl_h.at[c].at[s], idx_s)
        _ones_job(rows[0], deg_sh, idx_s, ssems)
        _ones_job(rows[0], cnt_sh, et_s, ssems, nch=HCH, base=base)
        plsc.subcore_barrier()
        _copy_out(deg_sh, deg_o.at[c], s, HALF)
        _copy_out(cnt_sh, cnt_o.at[c], s, RELS)
        _copy_out(sumr_sh, sumr_o.at[c], s, RELS)

    return k(h, src_i, dst_i, et_i, dstl_i, zeros_h, ones_h)


def _sc_agg(hw, erw, src_i, et_i, dstl_i, zeros_h, tok):
    out_type = jax.ShapeDtypeStruct((NC, HALF, H), jnp.float32)
    scratch = [
        pltpu.VMEM((CH, CHUNK), jnp.int32),
        pltpu.VMEM((CH, CHUNK), jnp.int32),
    ] + [pltpu.VMEM((CHUNK, H), jnp.float32)] * 2 + [
        pltpu.VMEM_SHARED((RPC, H), jnp.float32),
    ] + [pltpu.SemaphoreType.DMA] * 4

    @functools.partial(pl.kernel, out_type=out_type, mesh=_sc_mesh(),
                       scratch_types=scratch)
    def k(hw_h, erw_h, src_h, et_h, dstl_h, z_h, tok_h,
          agg_o,
          idx_s, dstl_s, *bufs):
        rows = list(bufs[:2])
        agg_sh = bufs[2]
        gsems = list(bufs[3:5])
        ssems = list(bufs[5:7])
        c = lax.axis_index("c")
        s = lax.axis_index("s")
        pltpu.sync_copy(z_h, rows[0])
        _zero_span(s, rows[0], agg_sh, RPC)
        pltpu.sync_copy(dstl_h.at[c].at[s], dstl_s)
        plsc.subcore_barrier()
        pltpu.sync_copy(src_h.at[s], idx_s)
        _stream_job(hw_h, idx_s, agg_sh, dstl_s, rows, gsems, ssems)
        pltpu.sync_copy(et_h.at[s], idx_s)
        _stream_job(erw_h, idx_s, agg_sh, dstl_s, rows, gsems, ssems)
        plsc.subcore_barrier()
        _copy_out(agg_sh, agg_o.at[c], s, HALF)

    return k(hw, erw, src_i, et_i, dstl_i, zeros_h, tok)


def kernel(edge_src, edge_dst, edge_type, dynamic_emb, emb_rel, w_ih, b_ih,
           w_hh, b_hh, time_gate_w, time_gate_b, w_neigh_0, loop_w_0,
           evolve_w_0, w_neigh_1, loop_w_1, evolve_w_1):
    pad = EPAD - E
    srcp = jnp.pad(edge_src, ((0, 0), (0, pad))).reshape(T, NS, CH, CHUNK)
    dstp = jnp.pad(edge_dst, ((0, 0), (0, pad)),
                   constant_values=NUM_ENTS).reshape(T, NS, CH, CHUNK)
    etp = jnp.pad(edge_type, ((0, 0), (0, pad)),
                  constant_values=2 * NUM_RELS).reshape(T, NS, CH, CHUNK)
    dstl = jnp.stack(
        [jnp.where((dstp >= c * HALF) & (dstp < (c + 1) * HALF),
                   dstp - c * HALF, JUNK) for c in range(NC)], axis=1)
    zeros_h = jnp.zeros((CHUNK, H), jnp.float32)
    ones_h = jnp.ones((CHUNK, H), jnp.float32)

    embp = jnp.pad(dynamic_emb, ((0, ROWS - NUM_ENTS), (0, 0)))
    erp = jnp.pad(emb_rel, ((0, RELS - 2 * NUM_RELS), (0, 0)))

    h, hw1, erw1, erw2 = _tc_prep(embp, w_neigh_0, w_neigh_1, erp)
    h0 = erp
    hist = []
    for t in range(T):
        deg, cnt, sumr = _sc_stage_h(
            h, srcp[t], dstp[t], etp[t], dstl[t], zeros_h, ones_h)
        tok1 = _tc_tok(deg[0, :8])
        agg1 = _sc_agg(hw1, erw1, srcp[t], etp[t], dstl[t], zeros_h, tok1)
        cur1, hw2 = _tc_mid(agg1, deg, h, loop_w_0, evolve_w_0, w_neigh_1)
        h0 = _tc_gru(sumr, cnt, erp, h0, w_ih, b_ih, w_hh, b_hh)
        tok2 = _tc_tok(agg1[0, :8])
        agg2 = _sc_agg(hw2, erw2, srcp[t], etp[t], dstl[t], zeros_h, tok2)
        h, hw1 = _tc_post(agg2, deg, cur1, h, loop_w_1, evolve_w_1,
                          time_gate_w, time_gate_b, w_neigh_0)
        hist.append(h[:NUM_ENTS])
    return jnp.stack(hist, axis=0), h0[:2 * NUM_RELS]


# speedup vs baseline: 1.5888x; 1.5012x over previous
import dataclasses
import functools

import jax
import jax.numpy as jnp
from jax import lax
from jax.experimental import pallas as pl
from jax.experimental.pallas import tpu as pltpu
from jax.experimental.pallas import tpu_sc as plsc

NUM_ENTS = 10000
NUM_RELS = 200
H = 128
T = 3
E = 320000
SLOPE = (1.0 / 8.0 + 1.0 / 3.0) / 2.0

NC = 2
NS = 16
CHUNK = 128
CH = 160
HCH = CH // 2
EPW = CH * CHUNK
EPAD = EPW * NS
ROWS = 10240
HALF = ROWS // 2
RPC = 5248
JUNK = HALF
RELS = 512
BLK = 2560


def _l2norm(x):
    n = jnp.sqrt(jnp.sum(x * x, axis=-1, keepdims=True))
    return x / jnp.maximum(n, 1e-12)


def _rrelu(x):
    return jnp.where(x >= 0, x, x * SLOPE)


def _dot(a, b):
    return lax.dot_general(a, b, (((1,), (0,)), ((), ())),
                           preferred_element_type=jnp.float32)


def _dot_t(a, b):
    return lax.dot_general(a, b, (((1,), (1,)), ((), ())),
                           preferred_element_type=jnp.float32)


def _map_sp(i):
    return (i // 2, i % 2, 0)


def _map_row(i):
    return (i, 0)


def _map0(i):
    return (0, 0)


def _tc_prep(emb, w0, w1, emb_rel):
    def body(emb_r, w0_r, w1_r, er_r, h_r, hw_r, e1_r, e2_r):
        h = _l2norm(emb_r[...])
        h_r[...] = h
        hw_r[...] = _dot(h, w0_r[...])
        er = er_r[...]
        e1_r[...] = _dot(er, w0_r[...])
        e2_r[...] = _dot(er, w1_r[...])

    return pl.pallas_call(
        body,
        grid=(ROWS // BLK,),
        in_specs=[pl.BlockSpec((BLK, H), _map_row),
                  pl.BlockSpec((H, H), _map0),
                  pl.BlockSpec((H, H), _map0),
                  pl.BlockSpec((RELS, H), _map0)],
        out_specs=[pl.BlockSpec((BLK, H), _map_row),
                   pl.BlockSpec((BLK, H), _map_row),
                   pl.BlockSpec((RELS, H), _map0),
                   pl.BlockSpec((RELS, H), _map0)],
        out_shape=[jax.ShapeDtypeStruct((ROWS, H), jnp.float32),
                   jax.ShapeDtypeStruct((ROWS, H), jnp.float32),
                   jax.ShapeDtypeStruct((RELS, H), jnp.float32),
                   jax.ShapeDtypeStruct((RELS, H), jnp.float32)],
    )(emb, w0, w1, emb_rel)


def _tc_mid(agg, deg, h, lw, ew, w_next):
    def body(ag_r, dg_r, h_r, lw_r, ew_r, wn_r, cur_r, hw_r):
        degv = dg_r[...][0, :, :1]
        norm = 1.0 / jnp.maximum(degv, 1.0)
        hv = h_r[...]
        lm = jnp.where(degv > 0, _dot(hv, lw_r[...]), _dot(hv, ew_r[...]))
        cur = _rrelu(ag_r[...][0] * norm + lm)
        cur_r[...] = cur
        hw_r[...] = _dot(cur, wn_r[...])

    return pl.pallas_call(
        body,
        grid=(ROWS // BLK,),
        in_specs=[pl.BlockSpec((1, BLK, H), _map_sp),
                  pl.BlockSpec((1, BLK, H), _map_sp),
                  pl.BlockSpec((BLK, H), _map_row),
                  pl.BlockSpec((H, H), _map0),
                  pl.BlockSpec((H, H), _map0),
                  pl.BlockSpec((H, H), _map0)],
        out_specs=[pl.BlockSpec((BLK, H), _map_row),
                   pl.BlockSpec((BLK, H), _map_row)],
        out_shape=[jax.ShapeDtypeStruct((ROWS, H), jnp.float32),
                   jax.ShapeDtypeStruct((ROWS, H), jnp.float32)],
    )(agg, deg, h, lw, ew, w_next)


def _tc_post(agg, deg, cur1, h, lw, ew, tgw, tgb, w0):
    def body(ag_r, dg_r, c1_r, h_r, lw_r, ew_r, tgw_r, tgb_r, w0_r,
             hn_r, hw_r):
        degv = dg_r[...][0, :, :1]
        norm = 1.0 / jnp.maximum(degv, 1.0)
        c1 = c1_r[...]
        lm = jnp.where(degv > 0, _dot(c1, lw_r[...]), _dot(c1, ew_r[...]))
        cur2 = _rrelu(ag_r[...][0] * norm + lm)
        ch = _l2norm(cur2)
        hv = h_r[...]
        tw = jax.nn.sigmoid(_dot(hv, tgw_r[...]) + tgb_r[...][None, :])
        hn = tw * ch + (1.0 - tw) * hv
        hn_r[...] = hn
        hw_r[...] = _dot(hn, w0_r[...])

    return pl.pallas_call(
        body,
        grid=(ROWS // BLK,),
        in_specs=[pl.BlockSpec((1, BLK, H), _map_sp),
                  pl.BlockSpec((1, BLK, H), _map_sp),
                  pl.BlockSpec((BLK, H), _map_row),
                  pl.BlockSpec((BLK, H), _map_row),
                  pl.BlockSpec((H, H), _map0),
                  pl.BlockSpec((H, H), _map0),
                  pl.BlockSpec((H, H), _map0),
                  pl.BlockSpec((H,), lambda i: (0,)),
                  pl.BlockSpec((H, H), _map0)],
        out_specs=[pl.BlockSpec((BLK, H), _map_row),
                   pl.BlockSpec((BLK, H), _map_row)],
        out_shape=[jax.ShapeDtypeStruct((ROWS, H), jnp.float32),
                   jax.ShapeDtypeStruct((ROWS, H), jnp.float32)],
    )(agg, deg, cur1, h, lw, ew, tgw, tgb, w0)


def _tc_gru(sumr, cnt, emb_rel, h0, w_ih, b_ih, w_hh, b_hh):
    def body(sr_r, ct_r, er_r, h0_r, wih_r, bih_r, whh_r, bhh_r, out_r):
        ctv = ct_r[...]
        c = 2.0 * (ctv[0, :, :1] + ctv[1, :, :1])
        srv = sr_r[...]
        sr = srv[0] + srv[1]
        x_in = jnp.where(c > 0, sr / jnp.maximum(c, 1.0), 0.0)
        er = er_r[...]
        wih = wih_r[...]
        gi = (_dot_t(er, wih[:, :H]) + _dot_t(x_in, wih[:, H:])
              + bih_r[...][None, :])
        h0v = h0_r[...]
        gh = _dot_t(h0v, whh_r[...]) + bhh_r[...][None, :]
        r = jax.nn.sigmoid(gi[:, :H] + gh[:, :H])
        z = jax.nn.sigmoid(gi[:, H:2 * H] + gh[:, H:2 * H])
        n = jnp.tanh(gi[:, 2 * H:] + r * gh[:, 2 * H:])
        out_r[...] = _l2norm((1.0 - z) * n + z * h0v)

    return pl.pallas_call(
        body,
        grid=(1,),
        in_specs=[pl.BlockSpec((NC, RELS, H), lambda i: (0, 0, 0)),
                  pl.BlockSpec((NC, RELS, H), lambda i: (0, 0, 0)),
                  pl.BlockSpec((RELS, H), _map0),
                  pl.BlockSpec((RELS, H), _map0),
                  pl.BlockSpec((3 * H, 2 * H), _map0),
                  pl.BlockSpec((3 * H,), lambda i: (0,)),
                  pl.BlockSpec((3 * H, H), _map0),
                  pl.BlockSpec((3 * H,), lambda i: (0,))],
        out_specs=pl.BlockSpec((RELS, H), _map0),
        out_shape=jax.ShapeDtypeStruct((RELS, H), jnp.float32),
    )(sumr, cnt, emb_rel, h0, w_ih, b_ih, w_hh, b_hh)


def _tc_tok(x):
    def body(x_r, o_r):
        o_r[...] = x_r[...] * 1.0

    return pl.pallas_call(
        body,
        grid=(1,),
        in_specs=[pl.BlockSpec((8, H), _map0)],
        out_specs=pl.BlockSpec((8, H), _map0),
        out_shape=jax.ShapeDtypeStruct((8, H), jnp.float32),
    )(x)


def _sc_mesh():
    return plsc.VectorSubcoreMesh(core_axis_name="c", subcore_axis_name="s")


def _sc_params():
    cp = pltpu.CompilerParams()
    if "needs_layout_passes" in pltpu.CompilerParams.__dataclass_fields__:
        cp = dataclasses.replace(cp, needs_layout_passes=False)
    return cp


def _zero_span(s, zbuf2d, tab_sh, total_rows):
    per = total_rows // NS
    base = s * per
    off = 0
    while off < per:
        n = min(CHUNK, per - off)
        pltpu.sync_copy(zbuf2d.at[pl.ds(0, n)],
                        tab_sh.at[pl.ds(base + off, n)])
        off += n


NBUF = 2
NG = 2


def _stream_job(tab_h, gidx, tgt_sh, sidx, rows, gsems, ssems,
                nch=CH, base=0):
    @pl.loop(0, nch, step=2)
    def _(i):
        j = base + i
        g0 = pltpu.async_copy(tab_h.at[gidx.at[j]], rows[0], gsems[0])
        g1 = pltpu.async_copy(tab_h.at[gidx.at[j + 1]], rows[1], gsems[1])
        g0.wait()
        s0 = pltpu.async_copy(rows[0], tgt_sh.at[sidx.at[j]], ssems[0],
                              add=True)
        g1.wait()
        s1 = pltpu.async_copy(rows[1], tgt_sh.at[sidx.at[j + 1]], ssems[1],
                              add=True)
        s0.wait()
        s1.wait()


def _reg_add(a, b):
    @pl.loop(0, CHUNK, step=2)
    def _(r):
        for rr in range(2):
            for q in range(8):
                sl = pl.ds(q * 16, 16)
                a[r + rr, sl] = a[r + rr, sl] + b[r + rr, sl]


def _merge_job(tabA, gidxA, tabB, gidxB, tgt_sh, sidx, rows, gsems, ssems,
               nch=CH, base=0):
    @pl.loop(0, nch, step=2)
    def _(i):
        j = base + i
        ga0 = pltpu.async_copy(tabA.at[gidxA.at[j]], rows[0], gsems[0])
        gb0 = pltpu.async_copy(tabB.at[gidxB.at[j]], rows[1], gsems[1])
        ga0.wait()
        gb0.wait()
        _reg_add(rows[0], rows[1])
        s0 = pltpu.async_copy(rows[0], tgt_sh.at[sidx.at[j]], ssems[0],
                              add=True)
        ga1 = pltpu.async_copy(tabA.at[gidxA.at[j + 1]], rows[2], gsems[0])
        gb1 = pltpu.async_copy(tabB.at[gidxB.at[j + 1]], rows[1], gsems[1])
        ga1.wait()
        gb1.wait()
        _reg_add(rows[2], rows[1])
        s1 = pltpu.async_copy(rows[2], tgt_sh.at[sidx.at[j + 1]], ssems[1],
                              add=True)
        s0.wait()
        s1.wait()


def _stream_job_g(tab_h, gidx, tgt_sh, sidx, rows, gsems, ssems, nb):
    @pl.loop(0, CH, step=2)
    def _(j):
        @pl.when(j < nb)
        def _():
            g0 = pltpu.async_copy(tab_h.at[gidx.at[j]], rows[0], gsems[0])
            g1 = pltpu.async_copy(tab_h.at[gidx.at[j + 1]], rows[1],
                                  gsems[1])
            g0.wait()
            s0 = pltpu.async_copy(rows[0], tgt_sh.at[sidx.at[j]], ssems[0],
                                  add=True)
            g1.wait()
            s1 = pltpu.async_copy(rows[1], tgt_sh.at[sidx.at[j + 1]],
                                  ssems[1], add=True)
            s0.wait()
            s1.wait()


def _ones_job_g(ones, tgt_sh, sidx, ssems, nb):
    @pl.loop(0, CH, step=NBUF)
    def _(j):
        @pl.when(j < nb)
        def _():
            ss = [pltpu.async_copy(ones, tgt_sh.at[sidx.at[j + k]],
                                   ssems[k], add=True)
                  for k in range(NBUF)]
            for s_ in ss:
                s_.wait()


def _ones_job(ones, tgt_sh, sidx, ssems, nch=CH, base=0):
    @pl.loop(0, nch, step=NBUF)
    def _(i):
        j = base + i
        ss = [pltpu.async_copy(ones, tgt_sh.at[sidx.at[j + k]], ssems[k],
                               add=True)
              for k in range(NBUF)]
        for s_ in ss:
            s_.wait()


def _my_count(nb_v, s):
    v = nb_v[...]
    lane = lax.iota(jnp.int32, NS)
    return jnp.sum(jnp.where(lane == s, v, 0), axis=0)


def _copy_out(sh, out_ref, s, total_rows):
    per = total_rows // NS
    pltpu.sync_copy(sh.at[pl.ds(s * per, per)],
                    out_ref.at[pl.ds(s * per, per)])


def _sc_stage_h(h, src_i, dst_i, et_i, dstl_i, cnts, zeros_h, ones_h):
    out_type = [
        jax.ShapeDtypeStruct((NC, HALF, H), jnp.float32),
        jax.ShapeDtypeStruct((NC, RELS, H), jnp.float32),
        jax.ShapeDtypeStruct((NC, RELS, H), jnp.float32),
    ]
    scratch = [
        pltpu.VMEM((CH, CHUNK), jnp.int32),
        pltpu.VMEM((CH, CHUNK), jnp.int32),
    ] + [pltpu.VMEM((CHUNK, H), jnp.float32)] * 2 + [
        pltpu.VMEM_SHARED((RPC, H), jnp.float32),
        pltpu.VMEM_SHARED((RELS, H), jnp.float32),
        pltpu.VMEM_SHARED((RELS, H), jnp.float32),
        pltpu.VMEM((NS,), jnp.int32),
    ] + [pltpu.SemaphoreType.DMA] * 4

    @functools.partial(pl.kernel, out_type=out_type, mesh=_sc_mesh(),
                       scratch_types=scratch, compiler_params=_sc_params())
    def k(h_h, src_h, dst_h, et_h, dstl_h, cnts_h, z_h, o_h,
          deg_o, cnt_o, sumr_o,
          idx_s, et_s, *bufs):
        rows = list(bufs[:2])
        deg_sh, cnt_sh, sumr_sh, nb_s = bufs[2:6]
        gsems = list(bufs[6:8])
        ssems = list(bufs[8:10])
        c = lax.axis_index("c")
        s = lax.axis_index("s")
        pltpu.sync_copy(z_h, rows[0])
        _zero_span(s, rows[0], deg_sh, RPC)
        _zero_span(s, rows[0], cnt_sh, RELS)
        _zero_span(s, rows[0], sumr_sh, RELS)
        pltpu.sync_copy(et_h.at[s], et_s)
        plsc.subcore_barrier()
        base = c * HCH
        pltpu.sync_copy(src_h.at[s], idx_s)
        _stream_job(h_h, idx_s, sumr_sh, et_s, rows, gsems, ssems,
                    nch=HCH, base=base)
        pltpu.sync_copy(dst_h.at[s], idx_s)
        _stream_job(h_h, idx_s, sumr_sh, et_s, rows, gsems, ssems,
                    nch=HCH, base=base)
        pltpu.sync_copy(o_h, rows[0])
        pltpu.sync_copy(cnts_h.at[c], nb_s)
        pltpu.sync_copy(dstl_h.at[c].at[s], idx_s)
        _ones_job_g(rows[0], deg_sh, idx_s, ssems, _my_count(nb_s, s))
        _ones_job(rows[0], cnt_sh, et_s, ssems, nch=HCH, base=base)
        plsc.subcore_barrier()
        _copy_out(deg_sh, deg_o.at[c], s, HALF)
        _copy_out(cnt_sh, cnt_o.at[c], s, RELS)
        _copy_out(sumr_sh, sumr_o.at[c], s, RELS)

    return k(h, src_i, dst_i, et_i, dstl_i, cnts, zeros_h, ones_h)


def _sc_agg(hw, erw, src_i, et_i, dstl_i, cnts, zeros_h, tok):
    out_type = jax.ShapeDtypeStruct((NC, HALF, H), jnp.float32)
    scratch = [
        pltpu.VMEM((CH, CHUNK), jnp.int32),
        pltpu.VMEM((CH, CHUNK), jnp.int32),
    ] + [pltpu.VMEM((CHUNK, H), jnp.float32)] * 2 + [
        pltpu.VMEM_SHARED((RPC, H), jnp.float32),
        pltpu.VMEM((NS,), jnp.int32),
    ] + [pltpu.SemaphoreType.DMA] * 4

    @functools.partial(pl.kernel, out_type=out_type, mesh=_sc_mesh(),
                       scratch_types=scratch, compiler_params=_sc_params())
    def k(hw_h, erw_h, src_h, et_h, dstl_h, cnts_h, z_h, tok_h,
          agg_o,
          idx_s, dstl_s, *bufs):
        rows = list(bufs[:2])
        agg_sh = bufs[2]
        nb_s = bufs[3]
        gsems = list(bufs[4:6])
        ssems = list(bufs[6:8])
        c = lax.axis_index("c")
        s = lax.axis_index("s")
        pltpu.sync_copy(z_h, rows[0])
        _zero_span(s, rows[0], agg_sh, RPC)
        pltpu.sync_copy(cnts_h.at[c], nb_s)
        pltpu.sync_copy(dstl_h.at[c].at[s], dstl_s)
        plsc.subcore_barrier()
        nb = _my_count(nb_s, s)
        pltpu.sync_copy(src_h.at[c].at[s], idx_s)
        _stream_job_g(hw_h, idx_s, agg_sh, dstl_s, rows, gsems, ssems, nb)
        pltpu.sync_copy(et_h.at[c].at[s], idx_s)
        _stream_job_g(erw_h, idx_s, agg_sh, dstl_s, rows, gsems, ssems, nb)
        plsc.subcore_barrier()
        _copy_out(agg_sh, agg_o.at[c], s, HALF)

    return k(hw, erw, src_i, et_i, dstl_i, cnts, zeros_h, tok)


def kernel(edge_src, edge_dst, edge_type, dynamic_emb, emb_rel, w_ih, b_ih,
           w_hh, b_hh, time_gate_w, time_gate_b, w_neigh_0, loop_w_0,
           evolve_w_0, w_neigh_1, loop_w_1, evolve_w_1):
    pad = EPAD - E
    srcf = jnp.pad(edge_src, ((0, 0), (0, pad)))
    dstf = jnp.pad(edge_dst, ((0, 0), (0, pad)), constant_values=NUM_ENTS)
    etf = jnp.pad(edge_type, ((0, 0), (0, pad)),
                  constant_values=2 * NUM_RELS)
    srcp = srcf.reshape(T, NS, CH, CHUNK)
    dstp = dstf.reshape(T, NS, CH, CHUNK)
    etp = etf.reshape(T, NS, CH, CHUNK)

    def rr(x):
        return x.reshape(CH, NS, CHUNK).transpose(1, 0, 2)

    srcC, etC, dstlC, cnts = [], [], [], []
    ar = jnp.arange(NS, dtype=jnp.int32)
    for t in range(T):
        order0 = jnp.argsort((dstf[t] >= HALF).astype(jnp.int32),
                             stable=True)
        s0 = srcf[t][order0]
        d0 = dstf[t][order0]
        e0 = etf[t][order0]
        s1, d1, e1 = s0[::-1], d0[::-1], e0[::-1]
        srcC.append(jnp.stack([rr(s0), rr(s1)]))
        etC.append(jnp.stack([rr(e0), rr(e1)]))
        dstlC.append(jnp.stack(
            [rr(jnp.where(d0 < HALF, d0, JUNK)),
             rr(jnp.where(d1 >= HALF, d1 - HALF, JUNK))]))
        nin0 = jnp.sum((dstf[t] < HALF).astype(jnp.int32))
        k0 = (nin0 + CHUNK - 1) // CHUNK
        k1 = (EPAD - nin0 + CHUNK - 1) // CHUNK
        cnts.append(jnp.stack(
            [jnp.maximum(0, (k0 - ar + NS - 1) // NS),
             jnp.maximum(0, (k1 - ar + NS - 1) // NS)]).astype(jnp.int32))
    zeros_h = jnp.zeros((CHUNK, H), jnp.float32)
    ones_h = jnp.ones((CHUNK, H), jnp.float32)

    embp = jnp.pad(dynamic_emb, ((0, ROWS - NUM_ENTS), (0, 0)))
    erp = jnp.pad(emb_rel, ((0, RELS - 2 * NUM_RELS), (0, 0)))

    h, hw1, erw1, erw2 = _tc_prep(embp, w_neigh_0, w_neigh_1, erp)
    h0 = erp
    hist = []
    for t in range(T):
        deg, cnt, sumr = _sc_stage_h(
            h, srcp[t], dstp[t], etp[t], dstlC[t], cnts[t], zeros_h, ones_h)
        tok1 = _tc_tok(deg[0, :8])
        agg1 = _sc_agg(hw1, erw1, srcC[t], etC[t], dstlC[t], cnts[t],
                       zeros_h, tok1)
        cur1, hw2 = _tc_mid(agg1, deg, h, loop_w_0, evolve_w_0, w_neigh_1)
        h0 = _tc_gru(sumr, cnt, erp, h0, w_ih, b_ih, w_hh, b_hh)
        tok2 = _tc_tok(agg1[0, :8])
        agg2 = _sc_agg(hw2, erw2, srcC[t], etC[t], dstlC[t], cnts[t],
                       zeros_h, tok2)
        h, hw1 = _tc_post(agg2, deg, cur1, h, loop_w_1, evolve_w_1,
                          time_gate_w, time_gate_b, w_neigh_0)
        hist.append(h[:NUM_ENTS])
    return jnp.stack(hist, axis=0), h0[:2 * NUM_RELS]
